# baseline (speedup 1.0000x reference)
import functools

import jax
import jax.numpy as jnp
from jax import lax
from jax.experimental import pallas as pl
from jax.experimental.pallas import tpu as pltpu
from jax.experimental.pallas import tpu_sc as plsc

N = 10000
E = 320000
NPAD = 10240
ROWS = NPAD // 128
NC = 2
NS = 16
NW = NC * NS
TCH = E // NS
INNER = NPAD // NS
B = 1024
ACT = 48


def _newton_rsqrt(d):
    i = plsc.bitcast(d, jnp.int32)
    i = jnp.full((16,), 0x5F3759DF, jnp.int32) - lax.shift_right_logical(i, 1)
    y = plsc.bitcast(i, jnp.float32)
    for _ in range(3):
        y = y * (1.5 - 0.5 * d * y * y)
    return y


def _make_sc_graph():
    mesh = plsc.VectorSubcoreMesh(core_axis_name="c", subcore_axis_name="s")

    @functools.partial(
        pl.kernel,
        out_type=(
            jax.ShapeDtypeStruct((NW, NPAD), jnp.float32),
            jax.ShapeDtypeStruct((NC, NPAD), jnp.float32),
            jax.ShapeDtypeStruct((NC, NPAD), jnp.float32),
            jax.ShapeDtypeStruct((NC, NPAD), jnp.float32),
            jax.ShapeDtypeStruct((NC, NS, NPAD), jnp.float32),
        ),
        mesh=mesh,
        compiler_params=pltpu.CompilerParams(needs_layout_passes=False),
        scratch_types=[
            pltpu.VMEM((TCH,), jnp.int32),
            pltpu.VMEM((TCH,), jnp.int32),
            pltpu.VMEM((NPAD,), jnp.float32),
            pltpu.VMEM((NPAD,), jnp.float32),
            pltpu.VMEM((NPAD,), jnp.float32),
            pltpu.VMEM((NS, INNER), jnp.float32),
            pltpu.VMEM((INNER,), jnp.float32),
            pltpu.VMEM((INNER,), jnp.float32),
            pltpu.SemaphoreType.DMA,
            pltpu.SemaphoreType.DMA,
            pltpu.SemaphoreType.DMA,
        ],
    )
    def sc_graph(dst_hbm, src_hbm,
                 uparts_hbm, dstage_hbm, wstage_hbm, vstage_hbm, parts_hbm,
                 dst_v, src_v, table_v, acc_v, acc2_v, mbuf_v, dslice_v,
                 oslice_v, sem_d, sem_s, sem_m):
        c = lax.axis_index("c")
        s = lax.axis_index("s")
        base = s * TCH
        cp_d = pltpu.async_copy(dst_hbm.at[pl.ds(base, TCH)], dst_v, sem_d)
        cp_s = pltpu.async_copy(src_hbm.at[pl.ds(base, TCH)], src_v, sem_s)

        zeros16 = jnp.zeros((16,), jnp.float32)

        def zero_acc():
            def zbody(i, carry):
                acc_v[pl.ds(i * 16, 16)] = zeros16
                acc2_v[pl.ds(i * 16, 16)] = zeros16
                return carry
            lax.fori_loop(0, NPAD // 16, zbody, 0, unroll=16)

        def publish_acc():
            def abody(i, carry):
                acc_v[pl.ds(i * 16, 16)] = (acc_v[pl.ds(i * 16, 16)]
                                            + acc2_v[pl.ds(i * 16, 16)])
                return carry
            lax.fori_loop(0, NPAD // 16, abody, 0, unroll=16)
            pltpu.sync_copy(acc_v, parts_hbm.at[c, s])

        def merge_read():
            cps = [pltpu.async_copy(
                parts_hbm.at[c, k, pl.ds(s * INNER, INNER)],
                mbuf_v.at[k], sem_m) for k in range(NS)]
            for cp in cps:
                cp.wait()

        def merged_chunk(q):
            m = mbuf_v[0, pl.ds(q * 16, 16)]
            for k in range(1, NS):
                m = m + mbuf_v[k, pl.ds(q * 16, 16)]
            return m

        zero_acc()

        cp_d.wait()
        ones16 = jnp.ones((16,), jnp.float32)

        @plsc.parallel_loop(0, TCH // 32, unroll=4)
        def _loop1(i):
            si0 = dst_v[pl.ds(i * 32, 16)]
            plsc.addupdate_scatter(acc_v, [si0], ones16)
            si1 = dst_v[pl.ds(i * 32 + 16, 16)]
            plsc.addupdate_scatter(acc2_v, [si1], ones16)

        publish_acc()
        plsc.subcore_barrier()

        merge_read()
        for q in range(INNER // 16):
            d = merged_chunk(q) + 1.0
            dslice_v[pl.ds(q * 16, 16)] = _newton_rsqrt(d)
        pltpu.sync_copy(dslice_v, dstage_hbm.at[c, pl.ds(s * INNER, INNER)])
        zero_acc()
        plsc.subcore_barrier()
        pltpu.sync_copy(dstage_hbm.at[c], table_v)

        cp_s.wait()

        @plsc.parallel_loop(0, TCH // 32, unroll=4)
        def _loop2(i):
            gi0 = dst_v[pl.ds(i * 32, 16)]
            val0 = plsc.load_gather(table_v, [gi0])
            si0 = src_v[pl.ds(i * 32, 16)]
            plsc.addupdate_scatter(acc_v, [si0], val0)
            gi1 = dst_v[pl.ds(i * 32 + 16, 16)]
            val1 = plsc.load_gather(table_v, [gi1])
            si1 = src_v[pl.ds(i * 32 + 16, 16)]
            plsc.addupdate_scatter(acc2_v, [si1], val1)

        publish_acc()
        plsc.subcore_barrier()

        merge_read()
        for q in range(INNER // 16):
            dv = dslice_v[pl.ds(q * 16, 16)]
            vv = dv * (merged_chunk(q) + dv)
            oslice_v[pl.ds(q * 16, 16)] = vv
        pltpu.sync_copy(oslice_v, vstage_hbm.at[c, pl.ds(s * INNER, INNER)])
        for q in range(INNER // 16):
            dv = dslice_v[pl.ds(q * 16, 16)]
            oslice_v[pl.ds(q * 16, 16)] = dv * oslice_v[pl.ds(q * 16, 16)]
        pltpu.sync_copy(oslice_v, wstage_hbm.at[c, pl.ds(s * INNER, INNER)])
        zero_acc()
        plsc.subcore_barrier()
        pltpu.sync_copy(wstage_hbm.at[c], table_v)

        @plsc.parallel_loop(0, TCH // 32, unroll=8)
        def _loop3(i):
            j = i + c * (TCH // 32)
            gi = dst_v[pl.ds(j * 16, 16)]
            val = plsc.load_gather(table_v, [gi])
            si = src_v[pl.ds(j * 16, 16)]
            plsc.addupdate_scatter(acc_v, [si], val)

        pltpu.sync_copy(acc_v, uparts_hbm.at[s * NC + c])

    return sc_graph


_sc_graph = _make_sc_graph()


def _head_body(parts_ref, dinv_ref, w_ref, v_ref, x3_ref,
               state_ref, action_ref,
               Wg1_ref, bg1_ref, Wg2_ref, bg2_ref,
               Wa0_ref, ba0_ref, Wa1_ref, ba1_ref, Wa2_ref, ba2_ref,
               Wc0_ref, bc0_ref, Wc1_ref, bc1_ref, Wc2_ref, bc2_ref,
               alp_ref, sval_ref, ent_ref):
    dinv = dinv_ref[...]
    w = w_ref[...]
    u = dinv * (jnp.sum(parts_ref[...], axis=0) + w)
    row = lax.broadcasted_iota(jnp.int32, (ROWS, 128), 0)
    col = lax.broadcasted_iota(jnp.int32, (ROWS, 128), 1)
    mask = (row * 128 + col) < N
    u = jnp.where(mask, u, 0.0)
    sv = jnp.sum(jnp.where(mask, v_ref[...], 0.0))
    t = jnp.sum(x3_ref[...] * u[:, :, None], axis=(0, 1)).reshape(1, 128)
    g1 = jnp.dot(t, Wg1_ref[...], preferred_element_type=jnp.float32) \
        + sv * bg1_ref[...]
    g = jnp.dot(g1, Wg2_ref[...], preferred_element_type=jnp.float32) / N \
        + bg2_ref[...]

    st = state_ref[...]

    def mlp(W0_ref, b0_ref, W1_ref, b1_ref):
        h = jnp.tanh(
            jnp.dot(st, W0_ref[0:128, :], preferred_element_type=jnp.float32)
            + jnp.dot(g, W0_ref[128:256, :], preferred_element_type=jnp.float32)
            + b0_ref[...])
        return jnp.tanh(
            jnp.dot(h, W1_ref[...], preferred_element_type=jnp.float32)
            + b1_ref[...])

    ya = mlp(Wa0_ref, ba0_ref, Wa1_ref, ba1_ref)
    logits = jnp.dot(ya, Wa2_ref[...], preferred_element_type=jnp.float32) \
        + ba2_ref[...]
    m = jnp.max(logits, axis=1, keepdims=True)
    ex = jnp.exp(logits - m)
    ssum = jnp.sum(ex, axis=1, keepdims=True)
    logp = logits - m - jnp.log(ssum)
    probs = ex / ssum
    onehot = lax.broadcasted_iota(jnp.int32, (B, ACT), 1) == action_ref[...]
    alp_ref[...] = jnp.sum(jnp.where(onehot, logp, 0.0), axis=1, keepdims=True)
    ent_ref[...] = -jnp.sum(probs * logp, axis=1, keepdims=True)

    yc = mlp(Wc0_ref, bc0_ref, Wc1_ref, bc1_ref)
    sval_ref[...] = jnp.dot(yc, Wc2_ref[...], preferred_element_type=jnp.float32) \
        + bc2_ref[...]


def kernel(state, action, x, edge_index, W_g1, b_g1, W_g2, b_g2,
           Wa0, ba0, Wa1, ba1, Wa2, ba2, Wc0, bc0, Wc1, bc1, Wc2, bc2):
    src = edge_index[0]
    dst = edge_index[1]

    u_parts, dinv_st, w_st, v_st, _ = _sc_graph(dst, src)

    x3 = jnp.pad(x, ((0, NPAD - N), (0, 0))).reshape(ROWS, 128, 128)
    action2 = action.astype(jnp.int32).reshape(B, 1)

    alp, sval, ent = pl.pallas_call(
        _head_body,
        out_shape=(jax.ShapeDtypeStruct((B, 1), jnp.float32),
                   jax.ShapeDtypeStruct((B, 1), jnp.float32),
                   jax.ShapeDtypeStruct((B, 1), jnp.float32)),
    )(u_parts.reshape(NW, ROWS, 128),
      dinv_st[0].reshape(ROWS, 128),
      w_st[0].reshape(ROWS, 128),
      v_st[0].reshape(ROWS, 128),
      x3, state, action2,
      W_g1, b_g1, W_g2, b_g2,
      Wa0, ba0, Wa1, ba1, Wa2, ba2,
      Wc0, bc0, Wc1, bc1, Wc2, bc2)

    return (alp[:, 0], sval, ent[:, 0])


# speedup vs baseline: 1.0929x; 1.0929x over previous
import functools

import jax
import jax.numpy as jnp
from jax import lax
from jax.experimental import pallas as pl
from jax.experimental.pallas import tpu as pltpu
from jax.experimental.pallas import tpu_sc as plsc

N = 10000
E = 320000
NPAD = 10240
ROWS = NPAD // 128
NC = 2
NS = 16
NW = NC * NS
TCH = E // NS
INNER = NPAD // NS
B = 1024
ACT = 48


def _newton_rsqrt(d):
    i = plsc.bitcast(d, jnp.int32)
    i = jnp.full((16,), 0x5F3759DF, jnp.int32) - lax.shift_right_logical(i, 1)
    y = plsc.bitcast(i, jnp.float32)
    for _ in range(3):
        y = y * (1.5 - 0.5 * d * y * y)
    return y


def _make_sc_graph():
    mesh = plsc.VectorSubcoreMesh(core_axis_name="c", subcore_axis_name="s")

    @functools.partial(
        pl.kernel,
        out_type=(
            jax.ShapeDtypeStruct((NW, NPAD), jnp.float32),
            jax.ShapeDtypeStruct((NC, NPAD), jnp.float32),
            jax.ShapeDtypeStruct((NC, NPAD), jnp.float32),
            jax.ShapeDtypeStruct((NC, NPAD), jnp.float32),
            jax.ShapeDtypeStruct((NC, NS, NPAD), jnp.float32),
        ),
        mesh=mesh,
        compiler_params=pltpu.CompilerParams(needs_layout_passes=False),
        scratch_types=[
            pltpu.VMEM((TCH,), jnp.int32),
            pltpu.VMEM((TCH,), jnp.int32),
            pltpu.VMEM((NPAD,), jnp.float32),
            pltpu.VMEM((NPAD,), jnp.float32),
            pltpu.VMEM((NS, INNER), jnp.float32),
            pltpu.VMEM((INNER,), jnp.float32),
            pltpu.VMEM((INNER,), jnp.float32),
            pltpu.SemaphoreType.DMA,
            pltpu.SemaphoreType.DMA,
            pltpu.SemaphoreType.DMA,
        ],
    )
    def sc_graph(dst_hbm, src_hbm,
                 uparts_hbm, dstage_hbm, wstage_hbm, vstage_hbm, parts_hbm,
                 dst_v, src_v, table_v, acc_v, mbuf_v, dslice_v, oslice_v,
                 sem_d, sem_s, sem_m):
        c = lax.axis_index("c")
        s = lax.axis_index("s")
        base = s * TCH
        cp_d = pltpu.async_copy(dst_hbm.at[pl.ds(base, TCH)], dst_v, sem_d)
        cp_s = pltpu.async_copy(src_hbm.at[pl.ds(base, TCH)], src_v, sem_s)

        zeros16 = jnp.zeros((16,), jnp.float32)

        def zero_acc():
            def zbody(i, carry):
                acc_v[pl.ds(i * 16, 16)] = zeros16
                return carry
            lax.fori_loop(0, NPAD // 16, zbody, 0, unroll=16)

        def publish_acc():
            pltpu.sync_copy(acc_v, parts_hbm.at[c, s])

        def merge_read():
            cps = [pltpu.async_copy(
                parts_hbm.at[c, k, pl.ds(s * INNER, INNER)],
                mbuf_v.at[k], sem_m) for k in range(NS)]
            for cp in cps:
                cp.wait()

        def merged_chunk(q):
            m = mbuf_v[0, pl.ds(q * 16, 16)]
            for k in range(1, NS):
                m = m + mbuf_v[k, pl.ds(q * 16, 16)]
            return m

        zero_acc()

        cp_d.wait()
        ones16 = jnp.ones((16,), jnp.float32)

        @plsc.parallel_loop(0, TCH // 16, unroll=8)
        def _loop1(i):
            si = dst_v[pl.ds(i * 16, 16)]
            plsc.addupdate_scatter(acc_v, [si], ones16)

        publish_acc()
        plsc.subcore_barrier()

        merge_read()
        for q in range(INNER // 16):
            d = merged_chunk(q) + 1.0
            dslice_v[pl.ds(q * 16, 16)] = _newton_rsqrt(d)
        pltpu.sync_copy(dslice_v, dstage_hbm.at[c, pl.ds(s * INNER, INNER)])
        zero_acc()
        plsc.subcore_barrier()
        pltpu.sync_copy(dstage_hbm.at[c], table_v)

        cp_s.wait()

        @plsc.parallel_loop(0, TCH // 16, unroll=8)
        def _loop2(i):
            gi = dst_v[pl.ds(i * 16, 16)]
            val = plsc.load_gather(table_v, [gi])
            si = src_v[pl.ds(i * 16, 16)]
            plsc.addupdate_scatter(acc_v, [si], val)

        publish_acc()
        plsc.subcore_barrier()

        merge_read()
        for q in range(INNER // 16):
            dv = dslice_v[pl.ds(q * 16, 16)]
            vv = dv * (merged_chunk(q) + dv)
            oslice_v[pl.ds(q * 16, 16)] = vv
        pltpu.sync_copy(oslice_v, vstage_hbm.at[c, pl.ds(s * INNER, INNER)])
        for q in range(INNER // 16):
            dv = dslice_v[pl.ds(q * 16, 16)]
            oslice_v[pl.ds(q * 16, 16)] = dv * oslice_v[pl.ds(q * 16, 16)]
        pltpu.sync_copy(oslice_v, wstage_hbm.at[c, pl.ds(s * INNER, INNER)])
        zero_acc()
        plsc.subcore_barrier()
        pltpu.sync_copy(wstage_hbm.at[c], table_v)

        @plsc.parallel_loop(0, TCH // 32, unroll=8)
        def _loop3(i):
            j = i + c * (TCH // 32)
            gi = dst_v[pl.ds(j * 16, 16)]
            val = plsc.load_gather(table_v, [gi])
            si = src_v[pl.ds(j * 16, 16)]
            plsc.addupdate_scatter(acc_v, [si], val)

        pltpu.sync_copy(acc_v, uparts_hbm.at[s * NC + c])

    return sc_graph


_sc_graph = _make_sc_graph()


def _head_body(parts_ref, dinv_ref, w_ref, v_ref, x3_ref,
               state_ref, action_ref,
               Wg1_ref, bg1_ref, Wg2_ref, bg2_ref,
               Wa0_ref, ba0_ref, Wa1_ref, ba1_ref, Wa2_ref, ba2_ref,
               Wc0_ref, bc0_ref, Wc1_ref, bc1_ref, Wc2_ref, bc2_ref,
               alp_ref, sval_ref, ent_ref):
    dinv = dinv_ref[...]
    w = w_ref[...]
    u = dinv * (jnp.sum(parts_ref[...], axis=0) + w)
    row = lax.broadcasted_iota(jnp.int32, (ROWS, 128), 0)
    col = lax.broadcasted_iota(jnp.int32, (ROWS, 128), 1)
    mask = (row * 128 + col) < N
    u = jnp.where(mask, u, 0.0)
    sv = jnp.sum(jnp.where(mask, v_ref[...], 0.0))
    t = jnp.sum(x3_ref[...] * u[:, :, None], axis=(0, 1)).reshape(1, 128)
    g1 = jnp.dot(t, Wg1_ref[...], preferred_element_type=jnp.float32) \
        + sv * bg1_ref[...]
    g = jnp.dot(g1, Wg2_ref[...], preferred_element_type=jnp.float32) / N \
        + bg2_ref[...]

    st = state_ref[...]

    def mlp(W0_ref, b0_ref, W1_ref, b1_ref):
        h = jnp.tanh(
            jnp.dot(st, W0_ref[0:128, :], preferred_element_type=jnp.float32)
            + jnp.dot(g, W0_ref[128:256, :], preferred_element_type=jnp.float32)
            + b0_ref[...])
        return jnp.tanh(
            jnp.dot(h, W1_ref[...], preferred_element_type=jnp.float32)
            + b1_ref[...])

    ya = mlp(Wa0_ref, ba0_ref, Wa1_ref, ba1_ref)
    logits = jnp.dot(ya, Wa2_ref[...], preferred_element_type=jnp.float32) \
        + ba2_ref[...]
    m = jnp.max(logits, axis=1, keepdims=True)
    ex = jnp.exp(logits - m)
    ssum = jnp.sum(ex, axis=1, keepdims=True)
    logp = logits - m - jnp.log(ssum)
    probs = ex / ssum
    onehot = lax.broadcasted_iota(jnp.int32, (B, ACT), 1) == action_ref[...]
    alp_ref[...] = jnp.sum(jnp.where(onehot, logp, 0.0), axis=1, keepdims=True)
    ent_ref[...] = -jnp.sum(probs * logp, axis=1, keepdims=True)

    yc = mlp(Wc0_ref, bc0_ref, Wc1_ref, bc1_ref)
    sval_ref[...] = jnp.dot(yc, Wc2_ref[...], preferred_element_type=jnp.float32) \
        + bc2_ref[...]


def kernel(state, action, x, edge_index, W_g1, b_g1, W_g2, b_g2,
           Wa0, ba0, Wa1, ba1, Wa2, ba2, Wc0, bc0, Wc1, bc1, Wc2, bc2):
    src = edge_index[0]
    dst = edge_index[1]

    u_parts, dinv_st, w_st, v_st, _ = _sc_graph(dst, src)

    x3 = jnp.pad(x, ((0, NPAD - N), (0, 0))).reshape(ROWS, 128, 128)
    action2 = action.astype(jnp.int32).reshape(B, 1)

    alp, sval, ent = pl.pallas_call(
        _head_body,
        out_shape=(jax.ShapeDtypeStruct((B, 1), jnp.float32),
                   jax.ShapeDtypeStruct((B, 1), jnp.float32),
                   jax.ShapeDtypeStruct((B, 1), jnp.float32)),
    )(u_parts.reshape(NW, ROWS, 128),
      dinv_st[0].reshape(ROWS, 128),
      w_st[0].reshape(ROWS, 128),
      v_st[0].reshape(ROWS, 128),
      x3, state, action2,
      W_g1, b_g1, W_g2, b_g2,
      Wa0, ba0, Wa1, ba1, Wa2, ba2,
      Wc0, bc0, Wc1, bc1, Wc2, bc2)

    return (alp[:, 0], sval, ent[:, 0])


# speedup vs baseline: 1.1042x; 1.0103x over previous
import functools

import jax
import jax.numpy as jnp
from jax import lax
from jax.experimental import pallas as pl
from jax.experimental.pallas import tpu as pltpu
from jax.experimental.pallas import tpu_sc as plsc

N = 10000
E = 320000
NPAD = 10240
ROWS = NPAD // 128
NC = 2
NS = 16
NW = NC * NS
TCH = E // NS
INNER = NPAD // NS
B = 1024
ACT = 48


def _newton_rsqrt(d):
    i = plsc.bitcast(d, jnp.int32)
    i = jnp.full((16,), 0x5F3759DF, jnp.int32) - lax.shift_right_logical(i, 1)
    y = plsc.bitcast(i, jnp.float32)
    for _ in range(3):
        y = y * (1.5 - 0.5 * d * y * y)
    return y


def _make_sc_graph():
    mesh = plsc.VectorSubcoreMesh(core_axis_name="c", subcore_axis_name="s")

    @functools.partial(
        pl.kernel,
        out_type=(
            jax.ShapeDtypeStruct((NW, NPAD), jnp.float32),
            jax.ShapeDtypeStruct((NC, NPAD), jnp.float32),
            jax.ShapeDtypeStruct((NC, NPAD), jnp.float32),
            jax.ShapeDtypeStruct((NC, NPAD), jnp.float32),
            jax.ShapeDtypeStruct((NC, NS, NPAD), jnp.float32),
        ),
        mesh=mesh,
        compiler_params=pltpu.CompilerParams(needs_layout_passes=False),
        scratch_types=[
            pltpu.VMEM((TCH,), jnp.int32),
            pltpu.VMEM((TCH,), jnp.int32),
            pltpu.VMEM((NPAD,), jnp.float32),
            pltpu.VMEM((NPAD,), jnp.float32),
            pltpu.VMEM((NS, INNER), jnp.float32),
            pltpu.VMEM((INNER,), jnp.float32),
            pltpu.VMEM((INNER,), jnp.float32),
            pltpu.SemaphoreType.DMA,
            pltpu.SemaphoreType.DMA,
            pltpu.SemaphoreType.DMA,
        ],
    )
    def sc_graph(dst_hbm, src_hbm,
                 uparts_hbm, dstage_hbm, wstage_hbm, vstage_hbm, parts_hbm,
                 dst_v, src_v, table_v, acc_v, mbuf_v, dslice_v, oslice_v,
                 sem_d, sem_s, sem_m):
        c = lax.axis_index("c")
        s = lax.axis_index("s")
        base = s * TCH
        cp_d = pltpu.async_copy(dst_hbm.at[pl.ds(base, TCH)], dst_v, sem_d)
        cp_s = pltpu.async_copy(src_hbm.at[pl.ds(base, TCH)], src_v, sem_s)

        zeros16 = jnp.zeros((16,), jnp.float32)

        def zero_acc():
            def zbody(i, carry):
                acc_v[pl.ds(i * 16, 16)] = zeros16
                return carry
            lax.fori_loop(0, NPAD // 16, zbody, 0, unroll=16)

        def publish_acc():
            pltpu.sync_copy(acc_v, parts_hbm.at[c, s])

        def merge_read():
            pltpu.sync_copy(
                parts_hbm.at[c, :, pl.ds(s * INNER, INNER)], mbuf_v)

        def merged_chunk(q):
            m = mbuf_v[0, pl.ds(q * 16, 16)]
            for k in range(1, NS):
                m = m + mbuf_v[k, pl.ds(q * 16, 16)]
            return m

        zero_acc()

        cp_d.wait()
        ones16 = jnp.ones((16,), jnp.float32)

        @plsc.parallel_loop(0, TCH // 16, unroll=8)
        def _loop1(i):
            si = dst_v[pl.ds(i * 16, 16)]
            plsc.addupdate_scatter(acc_v, [si], ones16)

        publish_acc()
        plsc.subcore_barrier()

        merge_read()
        for q in range(INNER // 16):
            d = merged_chunk(q) + 1.0
            dslice_v[pl.ds(q * 16, 16)] = _newton_rsqrt(d)
        pltpu.sync_copy(dslice_v, dstage_hbm.at[c, pl.ds(s * INNER, INNER)])
        zero_acc()
        plsc.subcore_barrier()
        pltpu.sync_copy(dstage_hbm.at[c], table_v)

        cp_s.wait()

        @plsc.parallel_loop(0, TCH // 16, unroll=8)
        def _loop2(i):
            gi = dst_v[pl.ds(i * 16, 16)]
            val = plsc.load_gather(table_v, [gi])
            si = src_v[pl.ds(i * 16, 16)]
            plsc.addupdate_scatter(acc_v, [si], val)

        publish_acc()
        plsc.subcore_barrier()

        merge_read()
        for q in range(INNER // 16):
            dv = dslice_v[pl.ds(q * 16, 16)]
            vv = dv * (merged_chunk(q) + dv)
            oslice_v[pl.ds(q * 16, 16)] = vv
        pltpu.sync_copy(oslice_v, vstage_hbm.at[c, pl.ds(s * INNER, INNER)])
        for q in range(INNER // 16):
            dv = dslice_v[pl.ds(q * 16, 16)]
            oslice_v[pl.ds(q * 16, 16)] = dv * oslice_v[pl.ds(q * 16, 16)]
        pltpu.sync_copy(oslice_v, wstage_hbm.at[c, pl.ds(s * INNER, INNER)])
        zero_acc()
        plsc.subcore_barrier()
        pltpu.sync_copy(wstage_hbm.at[c], table_v)

        @plsc.parallel_loop(0, TCH // 32, unroll=8)
        def _loop3(i):
            j = i + c * (TCH // 32)
            gi = dst_v[pl.ds(j * 16, 16)]
            val = plsc.load_gather(table_v, [gi])
            si = src_v[pl.ds(j * 16, 16)]
            plsc.addupdate_scatter(acc_v, [si], val)

        pltpu.sync_copy(acc_v, uparts_hbm.at[s * NC + c])

    return sc_graph


_sc_graph = _make_sc_graph()


def _head_body(parts_ref, dinv_ref, w_ref, v_ref, x3_ref,
               state_ref, action_ref,
               Wg1_ref, bg1_ref, Wg2_ref, bg2_ref,
               Wa0_ref, ba0_ref, Wa1_ref, ba1_ref, Wa2_ref, ba2_ref,
               Wc0_ref, bc0_ref, Wc1_ref, bc1_ref, Wc2_ref, bc2_ref,
               alp_ref, sval_ref, ent_ref):
    dinv = dinv_ref[...]
    w = w_ref[...]
    u = dinv * (jnp.sum(parts_ref[...], axis=0) + w)
    row = lax.broadcasted_iota(jnp.int32, (ROWS, 128), 0)
    col = lax.broadcasted_iota(jnp.int32, (ROWS, 128), 1)
    mask = (row * 128 + col) < N
    u = jnp.where(mask, u, 0.0)
    sv = jnp.sum(jnp.where(mask, v_ref[...], 0.0))
    t = jnp.sum(x3_ref[...] * u[:, :, None], axis=(0, 1)).reshape(1, 128)
    g1 = jnp.dot(t, Wg1_ref[...], preferred_element_type=jnp.float32) \
        + sv * bg1_ref[...]
    g = jnp.dot(g1, Wg2_ref[...], preferred_element_type=jnp.float32) / N \
        + bg2_ref[...]

    st = state_ref[...]

    def mlp(W0_ref, b0_ref, W1_ref, b1_ref):
        h = jnp.tanh(
            jnp.dot(st, W0_ref[0:128, :], preferred_element_type=jnp.float32)
            + jnp.dot(g, W0_ref[128:256, :], preferred_element_type=jnp.float32)
            + b0_ref[...])
        return jnp.tanh(
            jnp.dot(h, W1_ref[...], preferred_element_type=jnp.float32)
            + b1_ref[...])

    ya = mlp(Wa0_ref, ba0_ref, Wa1_ref, ba1_ref)
    logits = jnp.dot(ya, Wa2_ref[...], preferred_element_type=jnp.float32) \
        + ba2_ref[...]
    m = jnp.max(logits, axis=1, keepdims=True)
    ex = jnp.exp(logits - m)
    ssum = jnp.sum(ex, axis=1, keepdims=True)
    logp = logits - m - jnp.log(ssum)
    probs = ex / ssum
    onehot = lax.broadcasted_iota(jnp.int32, (B, ACT), 1) == action_ref[...]
    alp_ref[...] = jnp.sum(jnp.where(onehot, logp, 0.0), axis=1, keepdims=True)
    ent_ref[...] = -jnp.sum(probs * logp, axis=1, keepdims=True)

    yc = mlp(Wc0_ref, bc0_ref, Wc1_ref, bc1_ref)
    sval_ref[...] = jnp.dot(yc, Wc2_ref[...], preferred_element_type=jnp.float32) \
        + bc2_ref[...]


def kernel(state, action, x, edge_index, W_g1, b_g1, W_g2, b_g2,
           Wa0, ba0, Wa1, ba1, Wa2, ba2, Wc0, bc0, Wc1, bc1, Wc2, bc2):
    src = edge_index[0]
    dst = edge_index[1]

    u_parts, dinv_st, w_st, v_st, _ = _sc_graph(dst, src)

    x3 = jnp.pad(x, ((0, NPAD - N), (0, 0))).reshape(ROWS, 128, 128)
    action2 = action.astype(jnp.int32).reshape(B, 1)

    alp, sval, ent = pl.pallas_call(
        _head_body,
        out_shape=(jax.ShapeDtypeStruct((B, 1), jnp.float32),
                   jax.ShapeDtypeStruct((B, 1), jnp.float32),
                   jax.ShapeDtypeStruct((B, 1), jnp.float32)),
    )(u_parts.reshape(NW, ROWS, 128),
      dinv_st[0].reshape(ROWS, 128),
      w_st[0].reshape(ROWS, 128),
      v_st[0].reshape(ROWS, 128),
      x3, state, action2,
      W_g1, b_g1, W_g2, b_g2,
      Wa0, ba0, Wa1, ba1, Wa2, ba2,
      Wc0, bc0, Wc1, bc1, Wc2, bc2)

    return (alp[:, 0], sval, ent[:, 0])
